# baseline (speedup 1.0000x reference)
import functools

import jax
import jax.numpy as jnp
from jax import lax
from jax.experimental import pallas as pl
from jax.experimental.pallas import tpu as pltpu
from jax.experimental.pallas import tpu_sc as plsc

BATCH = 4096
MAXLEN = 200
EMBED = 64
HIDDEN = 128
CLASS_NUM = 100

_NC = 2
_NS = 16
_NW = _NC * _NS
_EPW = BATCH // _NW
_C0 = 128
_C1 = MAXLEN - _C0


def _pool_sc(idx_flat, table):
    mesh = plsc.VectorSubcoreMesh(core_axis_name="c", subcore_axis_name="s")

    @functools.partial(
        pl.kernel,
        out_type=jax.ShapeDtypeStruct((BATCH, EMBED), jnp.float32),
        mesh=mesh,
        scratch_types=[
            pltpu.VMEM((_EPW * MAXLEN,), jnp.int32),
            pltpu.VMEM((MAXLEN, EMBED), jnp.float32),
            pltpu.VMEM((_EPW, EMBED), jnp.float32),
            pltpu.SemaphoreType.DMA,
        ],
        compiler_params=pltpu.CompilerParams(use_tc_tiling_on_sc=False),
    )
    def k(idx_hbm, table_hbm, out_hbm, idx_v, buf_v, out_v, sem):
        wid = lax.axis_index("s") * _NC + lax.axis_index("c")
        base = wid * _EPW
        pltpu.sync_copy(idx_hbm.at[pl.ds(base * MAXLEN, _EPW * MAXLEN)], idx_v)

        scale = jnp.float32(1.0 / MAXLEN)

        @pl.loop(0, _EPW)
        def _elem(b):
            off = pl.multiple_of(b * MAXLEN, 8)
            d0 = pltpu.async_copy(
                table_hbm.at[idx_v.at[pl.ds(off, _C0)]],
                buf_v.at[pl.ds(0, _C0)], sem)
            d1 = pltpu.async_copy(
                table_hbm.at[idx_v.at[pl.ds(off + _C0, _C1)]],
                buf_v.at[pl.ds(_C0, _C1)], sem)
            d0.wait()
            d1.wait()

            zero = jnp.zeros((16,), jnp.float32)

            def red(r, accs):
                a0, a1, a2, a3 = accs
                a0 = a0 + buf_v[r, pl.ds(0, 16)]
                a1 = a1 + buf_v[r, pl.ds(16, 16)]
                a2 = a2 + buf_v[r, pl.ds(32, 16)]
                a3 = a3 + buf_v[r, pl.ds(48, 16)]
                return (a0, a1, a2, a3)

            a0, a1, a2, a3 = lax.fori_loop(0, MAXLEN, red,
                                           (zero, zero, zero, zero))
            out_v[b, pl.ds(0, 16)] = a0 * scale
            out_v[b, pl.ds(16, 16)] = a1 * scale
            out_v[b, pl.ds(32, 16)] = a2 * scale
            out_v[b, pl.ds(48, 16)] = a3 * scale

        pltpu.sync_copy(out_v, out_hbm.at[pl.ds(base, _EPW)])

    return k(idx_flat, table)


def _dense_body(x_ref, w1_ref, b1_ref, w2_ref, b2_ref, o_ref):
    x = x_ref[...]
    h = jnp.maximum(
        jnp.dot(x, w1_ref[...], preferred_element_type=jnp.float32)
        + b1_ref[...], 0.0)
    logits = (jnp.dot(h, w2_ref[...], preferred_element_type=jnp.float32)
              + b2_ref[...])
    m = jnp.max(logits, axis=-1, keepdims=True)
    e = jnp.exp(logits - m)
    o_ref[...] = e / jnp.sum(e, axis=-1, keepdims=True)


def _dense_tc(pooled, W1, b1, W2, b2):
    bm = 512
    grid = (BATCH // bm,)
    return pl.pallas_call(
        _dense_body,
        grid=grid,
        in_specs=[
            pl.BlockSpec((bm, EMBED), lambda i: (i, 0)),
            pl.BlockSpec((EMBED, HIDDEN), lambda i: (0, 0)),
            pl.BlockSpec((1, HIDDEN), lambda i: (0, 0)),
            pl.BlockSpec((HIDDEN, CLASS_NUM), lambda i: (0, 0)),
            pl.BlockSpec((1, CLASS_NUM), lambda i: (0, 0)),
        ],
        out_specs=pl.BlockSpec((bm, CLASS_NUM), lambda i: (i, 0)),
        out_shape=jax.ShapeDtypeStruct((BATCH, CLASS_NUM), jnp.float32),
    )(pooled, W1, b1.reshape(1, HIDDEN), W2, b2.reshape(1, CLASS_NUM))


def kernel(indices, table, W1, b1, W2, b2):
    idx_flat = indices.reshape(-1).astype(jnp.int32)
    pooled = _pool_sc(idx_flat, table)
    return _dense_tc(pooled, W1, b1, W2, b2)
